# Initial kernel scaffold; baseline (speedup 1.0000x reference)
import functools

import jax
import jax.numpy as jnp
from jax import lax
from jax.experimental import pallas as pl
from jax.experimental.pallas import tpu as pltpu
from jax.experimental.pallas import tpu_sc as plsc

_EMBED_DIM = 64


@functools.cache
def _build_gather(num_rows: int):
    try:
        info = plsc.get_sparse_core_info()
        num_cores, num_subcores = info.num_cores, info.num_subcores
    except Exception:
        num_cores, num_subcores = 2, 16
    num_workers = num_cores * num_subcores
    assert num_rows % num_workers == 0
    rows_per_worker = num_rows // num_workers
    chunk = 1024
    while rows_per_worker % chunk:
        chunk //= 2
    num_chunks = rows_per_worker // chunk

    mesh = plsc.VectorSubcoreMesh(core_axis_name="c", subcore_axis_name="s")

    @functools.partial(
        pl.kernel,
        out_type=jax.ShapeDtypeStruct((num_rows, _EMBED_DIM), jnp.float32),
        mesh=mesh,
        scratch_types=[
            pltpu.VMEM((chunk,), jnp.int32),
            pltpu.VMEM((chunk, _EMBED_DIM), jnp.float32),
            pltpu.SemaphoreType.DMA,
        ],
    )
    def gather(table_hbm, idx_hbm, out_hbm, idx_v, rows_v, sem):
        wid = lax.axis_index("s") * num_cores + lax.axis_index("c")
        first = wid * rows_per_worker

        def step(t, carry):
            base = first + t * chunk
            pltpu.sync_copy(idx_hbm.at[pl.ds(base, chunk)], idx_v)
            pltpu.async_copy(table_hbm.at[idx_v], rows_v, sem).wait()
            pltpu.sync_copy(rows_v, out_hbm.at[pl.ds(base, chunk)])
            return carry

        lax.fori_loop(0, num_chunks, step, 0)

    return gather


def kernel(token_ids, weight):
    idx = token_ids.reshape(-1).astype(jnp.int32)
    out = _build_gather(idx.shape[0])(weight, idx)
    return out.reshape(*token_ids.shape, _EMBED_DIM)


# speedup vs baseline: 1.8440x; 1.8440x over previous
import functools

import jax
import jax.numpy as jnp
from jax import lax
from jax.experimental import pallas as pl
from jax.experimental.pallas import tpu as pltpu
from jax.experimental.pallas import tpu_sc as plsc

_EMBED_DIM = 64


@functools.cache
def _build_gather(num_rows: int):
    try:
        info = plsc.get_sparse_core_info()
        num_cores, num_subcores = info.num_cores, info.num_subcores
    except Exception:
        num_cores, num_subcores = 2, 16
    num_workers = num_cores * num_subcores
    assert num_rows % num_workers == 0
    rows_per_worker = num_rows // num_workers
    chunk = 1024
    while rows_per_worker % chunk:
        chunk //= 2
    num_chunks = rows_per_worker // chunk

    mesh = plsc.VectorSubcoreMesh(core_axis_name="c", subcore_axis_name="s")

    @functools.partial(
        pl.kernel,
        out_type=jax.ShapeDtypeStruct((num_rows, _EMBED_DIM), jnp.float32),
        mesh=mesh,
        scratch_types=[
            pltpu.VMEM((chunk,), jnp.int32),
            pltpu.VMEM((chunk, _EMBED_DIM), jnp.float32),
            pltpu.SemaphoreType.DMA,
        ],
        compiler_params=pltpu.CompilerParams(use_tc_tiling_on_sc=False),
    )
    def gather(table_hbm, idx_hbm, out_hbm, idx_v, rows_v, sem):
        wid = lax.axis_index("s") * num_cores + lax.axis_index("c")
        first = wid * rows_per_worker

        def step(t, carry):
            base = first + t * chunk
            pltpu.sync_copy(idx_hbm.at[pl.ds(base, chunk)], idx_v)
            pltpu.async_copy(table_hbm.at[idx_v], rows_v, sem).wait()
            pltpu.sync_copy(rows_v, out_hbm.at[pl.ds(base, chunk)])
            return carry

        lax.fori_loop(0, num_chunks, step, 0)

    return gather


def kernel(token_ids, weight):
    idx = token_ids.reshape(-1).astype(jnp.int32)
    out = _build_gather(idx.shape[0])(weight, idx)
    return out.reshape(*token_ids.shape, _EMBED_DIM)


# speedup vs baseline: 1.8500x; 1.0033x over previous
import functools

import jax
import jax.numpy as jnp
from jax import lax
from jax.experimental import pallas as pl
from jax.experimental.pallas import tpu as pltpu
from jax.experimental.pallas import tpu_sc as plsc

_EMBED_DIM = 64
_CHUNK = 512
_NBUF = 2


@functools.cache
def _build_gather(num_rows: int):
    try:
        info = plsc.get_sparse_core_info()
        num_cores, num_subcores = info.num_cores, info.num_subcores
    except Exception:
        num_cores, num_subcores = 2, 16
    num_workers = num_cores * num_subcores
    assert num_rows % num_workers == 0
    rows_per_worker = num_rows // num_workers
    chunk, nbuf = _CHUNK, _NBUF
    assert rows_per_worker % (chunk * nbuf) == 0
    num_chunks = rows_per_worker // chunk
    num_blocks = num_chunks // nbuf
    lag = nbuf - 1

    mesh = plsc.VectorSubcoreMesh(core_axis_name="c", subcore_axis_name="s")

    @functools.partial(
        pl.kernel,
        out_type=jax.ShapeDtypeStruct((num_rows, _EMBED_DIM), jnp.float32),
        mesh=mesh,
        scratch_types=(
            [pltpu.VMEM((chunk,), jnp.int32) for _ in range(nbuf)]
            + [pltpu.VMEM((chunk, _EMBED_DIM), jnp.float32) for _ in range(nbuf)]
            + [pltpu.SemaphoreType.DMA for _ in range(2 * nbuf)]
        ),
        compiler_params=pltpu.CompilerParams(use_tc_tiling_on_sc=False),
    )
    def gather(table_hbm, idx_hbm, out_hbm, *scr):
        idx_v = scr[0:nbuf]
        rows_v = scr[nbuf : 2 * nbuf]
        sem_g = scr[2 * nbuf : 3 * nbuf]
        sem_w = scr[3 * nbuf : 4 * nbuf]

        wid = lax.axis_index("s") * num_cores + lax.axis_index("c")
        first = wid * rows_per_worker

        def fire_gather(t, b):
            pltpu.sync_copy(idx_hbm.at[pl.ds(first + t * chunk, chunk)], idx_v[b])
            pltpu.async_copy(table_hbm.at[idx_v[b]], rows_v[b], sem_g[b])

        def drain_gather_start_writeout(u, b):
            pltpu.make_async_copy(table_hbm.at[idx_v[b]], rows_v[b], sem_g[b]).wait()
            pltpu.async_copy(
                rows_v[b], out_hbm.at[pl.ds(first + u * chunk, chunk)], sem_w[b]
            )

        def wait_writeout(v, b):
            pltpu.make_async_copy(
                rows_v[b], out_hbm.at[pl.ds(first + v * chunk, chunk)], sem_w[b]
            ).wait()

        for t in range(nbuf):
            if t - lag >= 0:
                drain_gather_start_writeout(t - lag, (t - lag) % nbuf)
            fire_gather(t, t % nbuf)

        def block(j, carry):
            for b in range(nbuf):
                t = j * nbuf + b
                drain_gather_start_writeout(t - lag, (b + nbuf - lag) % nbuf)
                wait_writeout(t - nbuf, b)
                fire_gather(t, b)
            return carry

        lax.fori_loop(1, num_blocks, block, 0)

        for u in range(num_chunks - lag, num_chunks):
            drain_gather_start_writeout(u, u % nbuf)
        for v in range(num_chunks - nbuf, num_chunks):
            wait_writeout(v, v % nbuf)

    return gather


def kernel(token_ids, weight):
    idx = token_ids.reshape(-1).astype(jnp.int32)
    out = _build_gather(idx.shape[0])(weight, idx)
    return out.reshape(*token_ids.shape, _EMBED_DIM)
